# initial kernel scaffold (unmeasured)
import jax
import jax.numpy as jnp
from jax import lax
from jax.experimental import pallas as pl
from jax.experimental.pallas import tpu as pltpu

N_DEV = 8
E_LOCAL = 4
N_EXP = 32
CAP = 128


def _moe_body(xc_ref, ew_ref, x_ref, sw_ref, yc_ref, sh_ref,
              comm_ref, send_sems, recv_sems, credit_sem):
    my = lax.axis_index("i")
    left = lax.rem(my + N_DEV - 1, N_DEV)
    right = lax.rem(my + 1, N_DEV)

    barrier = pltpu.get_barrier_semaphore()
    for nbr in (left, right):
        pl.semaphore_signal(barrier, inc=1, device_id=(nbr,),
                            device_id_type=pl.DeviceIdType.MESH)
    pl.semaphore_wait(barrier, 2)

    def compute_group(g, w_ref):
        base = g * (E_LOCAL * CAP)
        for j in range(E_LOCAL):
            xg = xc_ref[pl.ds(base + j * CAP, CAP), :]
            yc_ref[pl.ds(base + j * CAP, CAP), :] = jnp.dot(
                xg, w_ref[j], preferred_element_type=jnp.float32)

    compute_group(my, ew_ref)
    sh_ref[...] = jnp.dot(x_ref[...], sw_ref[...],
                          preferred_element_type=jnp.float32)

    for h in range(N_DEV - 1):
        if h >= 2:
            pl.semaphore_wait(credit_sem, 1)
        src = ew_ref if h == 0 else comm_ref.at[h % 2]
        rdma = pltpu.make_async_remote_copy(
            src_ref=src,
            dst_ref=comm_ref.at[(h + 1) % 2],
            send_sem=send_sems.at[h % 2],
            recv_sem=recv_sems.at[(h + 1) % 2],
            device_id=(right,),
            device_id_type=pl.DeviceIdType.MESH,
        )
        rdma.start()
        rdma.wait()
        if 1 <= h <= N_DEV - 3:
            pl.semaphore_signal(credit_sem, inc=1, device_id=(left,),
                                device_id_type=pl.DeviceIdType.MESH)
        g = lax.rem(my + (N_DEV - 1 - h), N_DEV)
        compute_group(g, comm_ref.at[(h + 1) % 2])


def kernel(x, router_W, route_idx, expert_W, shared_W):
    n_tok, d = x.shape
    e_loc, _, h_dim = expert_W.shape

    scores = jnp.dot(x, router_W)
    probs = jax.nn.softmax(scores, axis=-1)
    p = jnp.take_along_axis(probs, route_idx, axis=1)[:, 0]
    e = route_idx[:, 0]

    order = jnp.argsort(e)
    sorted_e = e[order]
    starts = jnp.searchsorted(sorted_e, jnp.arange(N_EXP, dtype=sorted_e.dtype))
    rank = jnp.arange(n_tok) - starts[sorted_e]
    rank = jnp.minimum(rank, CAP - 1)
    slot_sorted = sorted_e * CAP + rank
    xc = jnp.zeros((N_EXP * CAP, d), x.dtype).at[slot_sorted].set(x[order])
    token_slot = jnp.zeros((n_tok,), jnp.int32).at[order].set(
        slot_sorted.astype(jnp.int32))

    yc, sh = pl.pallas_call(
        _moe_body,
        out_shape=[
            jax.ShapeDtypeStruct((N_EXP * CAP, h_dim), jnp.float32),
            jax.ShapeDtypeStruct((n_tok, h_dim), jnp.float32),
        ],
        in_specs=[pl.BlockSpec(memory_space=pltpu.VMEM)] * 4,
        out_specs=[pl.BlockSpec(memory_space=pltpu.VMEM)] * 2,
        scratch_shapes=[
            pltpu.VMEM((2, e_loc, d, h_dim), jnp.float32),
            pltpu.SemaphoreType.DMA((2,)),
            pltpu.SemaphoreType.DMA((2,)),
            pltpu.SemaphoreType.REGULAR,
        ],
        compiler_params=pltpu.CompilerParams(collective_id=0),
    )(xc, expert_W, x, shared_W)

    return sh + p[:, None] * yc[token_slot]


# baseline (device time: 994298 ns/iter reference)
import jax
import jax.numpy as jnp
from jax import lax
from jax.experimental import pallas as pl
from jax.experimental.pallas import tpu as pltpu

N_DEV = 8
E_LOCAL = 4
N_EXP = 32
CAP = 128


def _moe_body(xc_ref, ew_ref, x_ref, sw_ref, yc_ref, sh_ref,
              comm_ref, send_sems, recv_sems, credit_sem):
    my = lax.axis_index("i")
    left = lax.rem(my + N_DEV - 1, N_DEV)
    right = lax.rem(my + 1, N_DEV)

    barrier = pltpu.get_barrier_semaphore()
    for nbr in (left, right):
        pl.semaphore_signal(barrier, inc=1, device_id=(nbr,),
                            device_id_type=pl.DeviceIdType.MESH)
    pl.semaphore_wait(barrier, 2)

    def compute_group(g, w_ref):
        base = g * (E_LOCAL * CAP)
        for j in range(E_LOCAL):
            xg = xc_ref[pl.ds(base + j * CAP, CAP), :]
            yc_ref[pl.ds(base + j * CAP, CAP), :] = jnp.dot(
                xg, w_ref[j], preferred_element_type=jnp.float32)

    compute_group(my, ew_ref)
    sh_ref[...] = jnp.dot(x_ref[...], sw_ref[...],
                          preferred_element_type=jnp.float32)

    for h in range(N_DEV - 1):
        if h >= 2:
            pl.semaphore_wait(credit_sem, 1)
        src = ew_ref if h == 0 else comm_ref.at[h % 2]
        rdma = pltpu.make_async_remote_copy(
            src_ref=src,
            dst_ref=comm_ref.at[(h + 1) % 2],
            send_sem=send_sems.at[h % 2],
            recv_sem=recv_sems.at[(h + 1) % 2],
            device_id=(right,),
            device_id_type=pl.DeviceIdType.MESH,
        )
        rdma.start()
        rdma.wait()
        if 1 <= h <= N_DEV - 3:
            pl.semaphore_signal(credit_sem, inc=1, device_id=(left,),
                                device_id_type=pl.DeviceIdType.MESH)
        g = lax.rem(my + (N_DEV - 1 - h), N_DEV)
        compute_group(g, comm_ref.at[(h + 1) % 2])


def kernel(x, router_W, route_idx, expert_W, shared_W):
    n_tok, d = x.shape
    e_loc, _, h_dim = expert_W.shape

    scores = jnp.dot(x, router_W)
    probs = jax.nn.softmax(scores, axis=-1)
    p = jnp.take_along_axis(probs, route_idx, axis=1)[:, 0]
    e = route_idx[:, 0]

    order = jnp.argsort(e)
    sorted_e = e[order]
    starts = jnp.searchsorted(sorted_e, jnp.arange(N_EXP, dtype=sorted_e.dtype))
    rank = jnp.arange(n_tok) - starts[sorted_e]
    rank = jnp.minimum(rank, CAP - 1)
    slot_sorted = sorted_e * CAP + rank
    xc = jnp.zeros((N_EXP * CAP, d), x.dtype).at[slot_sorted].set(x[order])
    token_slot = jnp.zeros((n_tok,), jnp.int32).at[order].set(
        slot_sorted.astype(jnp.int32))

    yc, sh = pl.pallas_call(
        _moe_body,
        out_shape=[
            jax.ShapeDtypeStruct((N_EXP * CAP, h_dim), jnp.float32),
            jax.ShapeDtypeStruct((n_tok, h_dim), jnp.float32),
        ],
        in_specs=[pl.BlockSpec(memory_space=pltpu.VMEM)] * 4,
        out_specs=[pl.BlockSpec(memory_space=pltpu.VMEM)] * 2,
        scratch_shapes=[
            pltpu.VMEM((2, e_loc, d, h_dim), jnp.float32),
            pltpu.SemaphoreType.DMA((2,)),
            pltpu.SemaphoreType.DMA((2,)),
            pltpu.SemaphoreType.REGULAR,
        ],
        compiler_params=pltpu.CompilerParams(
            collective_id=0,
            vmem_limit_bytes=100 * 1024 * 1024,
        ),
    )(xc, expert_W, x, shared_W)

    return sh + p[:, None] * yc[token_slot]
